# baseline (speedup 1.0000x reference)
import functools

import jax
import jax.numpy as jnp
from jax import lax
from jax.experimental import pallas as pl
from jax.experimental.pallas import tpu as pltpu
from jax.experimental.pallas import tpu_sc as plsc

N_NODES = 100000
TYPE_NUM = 300
DIM = 128

NC = 2
NS = 16
LANES = 16
NW = NC * NS

CHUNK = 128
N_CHUNKS = 25
PER_W = CHUNK * N_CHUNKS
N_PAD = PER_W * NW


def _body(nt_hbm, ls_hbm, w_hbm, out_hbm, nt_v, ls_v, idx_v, rows_v, sem):
    wid = lax.axis_index("s") * NC + lax.axis_index("c")
    base = wid * PER_W

    pltpu.sync_copy(nt_hbm.at[pl.ds(base, PER_W)], nt_v)
    pltpu.sync_copy(ls_hbm.at[pl.ds(base, PER_W)], ls_v)

    def compute_idx(j, _):
        def inner(k, _):
            off = j * CHUNK + k * LANES
            nt = nt_v[pl.ds(off, LANES)]
            l = ls_v[pl.ds(off, LANES)]
            idx_v[j, pl.ds(k * LANES, LANES)] = nt + l * 100
            return 0
        return lax.fori_loop(0, CHUNK // LANES, inner, 0)

    lax.fori_loop(0, N_CHUNKS, compute_idx, 0)

    def do_chunk(j, _):
        cp = pltpu.make_async_copy(w_hbm.at[idx_v.at[j]], rows_v, sem)
        cp.start()
        cp.wait()
        pltpu.sync_copy(rows_v, out_hbm.at[pl.ds(base + j * CHUNK, CHUNK)])
        return 0

    lax.fori_loop(0, N_CHUNKS, do_chunk, 0)


_sc_gather = functools.partial(
    pl.kernel,
    mesh=plsc.VectorSubcoreMesh(core_axis_name="c", subcore_axis_name="s"),
    out_type=jax.ShapeDtypeStruct((N_PAD, DIM), jnp.float32),
    scratch_types=[
        pltpu.VMEM((PER_W,), jnp.int32),
        pltpu.VMEM((PER_W,), jnp.int32),
        pltpu.VMEM((N_CHUNKS, CHUNK), jnp.int32),
        pltpu.VMEM((CHUNK, DIM), jnp.float32),
        pltpu.SemaphoreType.DMA,
    ],
)(_body)


def kernel(node_type, ls, weight):
    pad = N_PAD - N_NODES
    nt = jnp.pad(node_type, (0, pad))
    lsp = jnp.pad(ls, (0, pad))
    out = _sc_gather(nt, lsp, weight)
    return out[:N_NODES]


# speedup vs baseline: 1.0881x; 1.0881x over previous
import functools

import jax
import jax.numpy as jnp
from jax import lax
from jax.experimental import pallas as pl
from jax.experimental.pallas import tpu as pltpu
from jax.experimental.pallas import tpu_sc as plsc

N_NODES = 100000
TYPE_NUM = 300
DIM = 128

NC = 2
NS = 16
LANES = 16
NW = NC * NS

CHUNK = 128
N_CHUNKS = 25
PER_W = CHUNK * N_CHUNKS
N_PAD = PER_W * NW


NBUF = 4


def _body(nt_hbm, ls_hbm, w_hbm, out_hbm, nt_v, ls_v, idx_v, rows_v,
          sem_g, sem_w):
    wid = lax.axis_index("s") * NC + lax.axis_index("c")
    base = wid * PER_W

    pltpu.sync_copy(nt_hbm.at[pl.ds(base, PER_W)], nt_v)
    pltpu.sync_copy(ls_hbm.at[pl.ds(base, PER_W)], ls_v)

    def compute_idx(j, _):
        def inner(k, _):
            off = j * CHUNK + k * LANES
            nt = nt_v[pl.ds(off, LANES)]
            l = ls_v[pl.ds(off, LANES)]
            idx_v[j, pl.ds(k * LANES, LANES)] = nt + l * 100
            return 0
        return lax.fori_loop(0, CHUNK // LANES, inner, 0)

    lax.fori_loop(0, N_CHUNKS, compute_idx, 0)

    def start_gather(j):
        pltpu.make_async_copy(
            w_hbm.at[idx_v.at[j]], rows_v.at[j % NBUF], sem_g).start()

    def start_write(j):
        pltpu.make_async_copy(
            rows_v.at[j % NBUF],
            out_hbm.at[pl.ds(base + j * CHUNK, CHUNK)], sem_w).start()

    def wait_gather(j):
        pltpu.make_async_copy(
            w_hbm.at[idx_v.at[j]], rows_v.at[j % NBUF], sem_g).wait()

    def wait_write(j):
        pltpu.make_async_copy(
            rows_v.at[j % NBUF],
            out_hbm.at[pl.ds(base + j * CHUNK, CHUNK)], sem_w).wait()

    waited_w = 0
    for j in range(min(NBUF - 1, N_CHUNKS)):
        start_gather(j)
    for j in range(N_CHUNKS):
        wait_gather(j)
        start_write(j)
        nxt = j + NBUF - 1
        if nxt < N_CHUNKS:
            if j >= 1:
                wait_write(waited_w)
                waited_w += 1
            start_gather(nxt)
    for j in range(waited_w, N_CHUNKS):
        wait_write(j)


_sc_gather = functools.partial(
    pl.kernel,
    mesh=plsc.VectorSubcoreMesh(core_axis_name="c", subcore_axis_name="s"),
    out_type=jax.ShapeDtypeStruct((N_PAD, DIM), jnp.float32),
    scratch_types=[
        pltpu.VMEM((PER_W,), jnp.int32),
        pltpu.VMEM((PER_W,), jnp.int32),
        pltpu.VMEM((N_CHUNKS, CHUNK), jnp.int32),
        pltpu.VMEM((NBUF, CHUNK, DIM), jnp.float32),
        pltpu.SemaphoreType.DMA,
        pltpu.SemaphoreType.DMA,
    ],
)(_body)


def kernel(node_type, ls, weight):
    pad = N_PAD - N_NODES
    nt = jnp.pad(node_type, (0, pad))
    lsp = jnp.pad(ls, (0, pad))
    out = _sc_gather(nt, lsp, weight)
    return out[:N_NODES]
